# baseline (speedup 1.0000x reference)
import jax
import jax.numpy as jnp
from jax.experimental import pallas as pl
from jax.experimental.pallas import tpu as pltpu


_TBLK = 1024
_K = 4


def _body(tok_hbm, tab_hbm, out_hbm, tab_v, in_v, out_v, tab_sem, in_sems, out_sems):
    n_chunks = tok_hbm.shape[0] // _TBLK
    tab_chunks = tab_hbm.shape[0] // _TBLK

    def in_copy(i, slot):
        return pltpu.make_async_copy(
            tok_hbm.at[pl.ds(i * _TBLK, _TBLK), :], in_v.at[slot], in_sems.at[slot])

    def out_copy(i, slot):
        return pltpu.make_async_copy(
            out_v.at[slot], out_hbm.at[pl.ds(i * _TBLK, _TBLK), :], out_sems.at[slot])

    pltpu.make_async_copy(tab_hbm, tab_v, tab_sem).start()

    for s in range(_K):
        in_copy(s, s).start()

    pltpu.make_async_copy(tab_hbm, tab_v, tab_sem).wait()

    def step(i, _):
        slot = jax.lax.rem(i, _K)
        in_copy(i, slot).wait()
        t = jax.lax.rem(i, tab_chunks) * _TBLK
        out_v[slot] = in_v[slot] + tab_v[pl.ds(t, _TBLK), :]
        out_copy(i, slot).start()

        @pl.when(i + _K < n_chunks)
        def _():
            in_copy(i + _K, slot).start()

        @pl.when(i >= _K - 1)
        def _():
            j = i - (_K - 1)
            out_copy(j, jax.lax.rem(j, _K)).wait()
        return 0

    jax.lax.fori_loop(0, n_chunks, step, 0)

    for r in range(_K - 1):
        idx = n_chunks - (_K - 1) + r
        out_copy(idx, idx % _K).wait()


def kernel(encoded_tokens, pos_table):
    batch, num_tokens, embed_dim = encoded_tokens.shape
    flat = encoded_tokens.reshape(batch * num_tokens, embed_dim)
    out = pl.pallas_call(
        _body,
        in_specs=[
            pl.BlockSpec(memory_space=pl.ANY),
            pl.BlockSpec(memory_space=pl.ANY),
        ],
        out_specs=pl.BlockSpec(memory_space=pl.ANY),
        out_shape=jax.ShapeDtypeStruct(flat.shape, flat.dtype),
        scratch_shapes=[
            pltpu.VMEM((num_tokens, embed_dim), jnp.float32),
            pltpu.VMEM((_K, _TBLK, embed_dim), jnp.float32),
            pltpu.VMEM((_K, _TBLK, embed_dim), jnp.float32),
            pltpu.SemaphoreType.DMA,
            pltpu.SemaphoreType.DMA((_K,)),
            pltpu.SemaphoreType.DMA((_K,)),
        ],
    )(flat, pos_table)
    return out.reshape(batch, num_tokens, embed_dim)
